# baseline (speedup 1.0000x reference)
import jax
import jax.numpy as jnp
from jax.experimental import pallas as pl
from jax.experimental.pallas import tpu as pltpu

_L = 8
_NCHAIN = 32 // _L


def _split3(z):
    hi = z.astype(jnp.bfloat16)
    r1 = z - hi.astype(jnp.float32)
    mid = r1.astype(jnp.bfloat16)
    lo = (r1 - mid.astype(jnp.float32)).astype(jnp.bfloat16)
    return hi, mid, lo


def _split2(z):
    hi = z.astype(jnp.bfloat16)
    lo = (z - hi.astype(jnp.float32)).astype(jnp.bfloat16)
    return hi, lo


def _dot(a, b):
    return jax.lax.dot_general(a, b, (((1,), (0,)), ((), ())),
                               preferred_element_type=jnp.float32)


def _body(a_ref, low_ref, high_ref, fa_ref, r_ref, e_ref, modd_ref, o_ref):
    a = a_ref[...]
    low = low_ref[...]
    high = high_ref[...]
    scale = 2.0 / (high - low)
    x = (a - low) * scale + 1.0
    x = jnp.clip(x, -1.0, 1.0)

    xh, xm, xl = _split3(x)
    xr = _dot(jnp.concatenate([xh, xm, xl], axis=1), r_ref[...])

    base = xr * fa_ref[...]
    s = jnp.sin(base)
    c = jnp.cos(base)

    parts = []
    for li in range(4):
        if li > 0:
            for _ in range(2):
                s, c = 2.0 * s * c, 2.0 * c * c - 1.0
        parts.append((s, c))

    E = e_ref[...]
    s_terms = []
    c_terms = []
    for sv, cv in parts:
        sh, sl = _split2(sv)
        ch, cl = _split2(cv)
        s_terms += [sh, sl]
        c_terms += [ch, cl]
    S = _dot(jnp.concatenate(s_terms, axis=1), E)
    C = _dot(jnp.concatenate(c_terms, axis=1), E)

    modd = modd_ref[...]
    act = modd == 1
    sd = 2.0 * S * C
    cd = 2.0 * C * C - 1.0
    S = jnp.where(act, sd, S)
    C = jnp.where(act, cd, C)

    o_ref[:, 0:2048] = S
    o_ref[:, 2048:4096] = C


def kernel(action, action_low, action_high, freqs):
    B, A = action.shape
    F = freqs.shape[0]
    P = A * _NCHAIN
    BB = 512

    low2 = action_low.reshape(1, A)
    high2 = action_high.reshape(1, A)
    fa = jnp.tile(freqs[::_L], (A,)).reshape(1, P)

    a_idx = jnp.arange(A)[:, None]
    p_idx = jnp.arange(P)[None, :]
    R1 = (p_idx // _NCHAIN == a_idx).astype(jnp.bfloat16)
    R = jnp.concatenate([R1, R1, R1], axis=0)

    r_li = (jnp.arange(2048) // 512)[:, None]
    r_p = (jnp.arange(2048) % 256)[:, None]
    l_idx = jnp.arange(A * F)[None, :]
    E = ((l_idx // _L == r_p) & ((l_idx % _L) // 2 == r_li)).astype(jnp.bfloat16)
    modd = (jnp.arange(A * F, dtype=jnp.int32) % 2).reshape(1, A * F)

    out = pl.pallas_call(
        _body,
        grid=(B // BB,),
        in_specs=[
            pl.BlockSpec((BB, A), lambda i: (i, 0)),
            pl.BlockSpec((1, A), lambda i: (0, 0)),
            pl.BlockSpec((1, A), lambda i: (0, 0)),
            pl.BlockSpec((1, P), lambda i: (0, 0)),
            pl.BlockSpec((3 * A, P), lambda i: (0, 0)),
            pl.BlockSpec((2048, A * F), lambda i: (0, 0)),
            pl.BlockSpec((1, A * F), lambda i: (0, 0)),
        ],
        out_specs=pl.BlockSpec((BB, 2 * A * F), lambda i: (i, 0)),
        out_shape=jax.ShapeDtypeStruct((B, 2 * A * F), jnp.float32),
        compiler_params=pltpu.CompilerParams(
            dimension_semantics=("parallel",),
        ),
    )(action, low2, high2, fa, R, E, modd)

    return out


# speedup vs baseline: 1.1875x; 1.1875x over previous
import jax
import jax.numpy as jnp
from jax.experimental import pallas as pl
from jax.experimental.pallas import tpu as pltpu

_L = 8
_NCHAIN = 32 // _L


def _split3(z):
    hi = z.astype(jnp.bfloat16)
    r1 = z - hi.astype(jnp.float32)
    mid = r1.astype(jnp.bfloat16)
    lo = (r1 - mid.astype(jnp.float32)).astype(jnp.bfloat16)
    return hi, mid, lo


def _split2(z):
    hi = z.astype(jnp.bfloat16)
    lo = (z - hi.astype(jnp.float32)).astype(jnp.bfloat16)
    return hi, lo


def _dot(a, b):
    return jax.lax.dot_general(a, b, (((1,), (0,)), ((), ())),
                               preferred_element_type=jnp.float32)


def _body(a_ref, low_ref, high_ref, fa_ref, r_ref, e_ref, kmod_ref, o_ref):
    a = a_ref[...]
    low = low_ref[...]
    high = high_ref[...]
    scale = 2.0 / (high - low)
    x = (a - low) * scale + 1.0
    x = jnp.clip(x, -1.0, 1.0)

    xh, xm, xl = _split3(x)
    xr = _dot(jnp.concatenate([xh, xm, xl], axis=1), r_ref[...])

    base = xr * fa_ref[...]
    s0 = jnp.sin(base)
    c0 = jnp.cos(base)

    s4, c4 = s0, c0
    for _ in range(4):
        s4, c4 = 2.0 * s4 * c4, 2.0 * c4 * c4 - 1.0

    E2 = e_ref[...]

    def expand(z):
        hi, lo = _split2(z)
        return _dot(jnp.concatenate([hi, lo], axis=1), E2)

    S0 = expand(s0)
    C0 = expand(c0)
    S4 = expand(s4)
    C4 = expand(c4)

    kmod = kmod_ref[...]
    k4 = jnp.where(kmod >= 4, kmod - 4, kmod)
    S = jnp.where(kmod >= 4, S4, S0)
    C = jnp.where(kmod >= 4, C4, C0)
    for j in range(1, 4):
        sd = 2.0 * S * C
        cd = 2.0 * C * C - 1.0
        act = k4 >= j
        S = jnp.where(act, sd, S)
        C = jnp.where(act, cd, C)

    o_ref[:, 0:2048] = S
    o_ref[:, 2048:4096] = C


def kernel(action, action_low, action_high, freqs):
    B, A = action.shape
    F = freqs.shape[0]
    P = A * _NCHAIN
    BB = 256

    low2 = action_low.reshape(1, A)
    high2 = action_high.reshape(1, A)
    fa = jnp.tile(freqs[::_L], (A,)).reshape(1, P)

    a_idx = jnp.arange(A)[:, None]
    p_idx = jnp.arange(P)[None, :]
    R1 = (p_idx // _NCHAIN == a_idx).astype(jnp.bfloat16)
    R = jnp.concatenate([R1, R1, R1], axis=0)

    pp_idx = jnp.arange(P)[:, None]
    l_idx = jnp.arange(A * F)[None, :]
    E1 = (l_idx // _L == pp_idx).astype(jnp.bfloat16)
    E = jnp.concatenate([E1, E1], axis=0)
    kmod = (jnp.arange(A * F, dtype=jnp.int32) % _L).reshape(1, A * F)

    out = pl.pallas_call(
        _body,
        grid=(B // BB,),
        in_specs=[
            pl.BlockSpec((BB, A), lambda i: (i, 0)),
            pl.BlockSpec((1, A), lambda i: (0, 0)),
            pl.BlockSpec((1, A), lambda i: (0, 0)),
            pl.BlockSpec((1, P), lambda i: (0, 0)),
            pl.BlockSpec((3 * A, P), lambda i: (0, 0)),
            pl.BlockSpec((2 * P, A * F), lambda i: (0, 0)),
            pl.BlockSpec((1, A * F), lambda i: (0, 0)),
        ],
        out_specs=pl.BlockSpec((BB, 2 * A * F), lambda i: (i, 0)),
        out_shape=jax.ShapeDtypeStruct((B, 2 * A * F), jnp.float32),
        compiler_params=pltpu.CompilerParams(
            dimension_semantics=("parallel",),
        ),
    )(action, low2, high2, fa, R, E, kmod)

    return out


# speedup vs baseline: 1.2564x; 1.0581x over previous
import jax
import jax.numpy as jnp
from jax.experimental import pallas as pl
from jax.experimental.pallas import tpu as pltpu

_L = 8
_NCHAIN = 32 // _L


def _split3(z):
    hi = z.astype(jnp.bfloat16)
    r1 = z - hi.astype(jnp.float32)
    mid = r1.astype(jnp.bfloat16)
    lo = (r1 - mid.astype(jnp.float32)).astype(jnp.bfloat16)
    return hi, mid, lo


def _split2(z):
    hi = z.astype(jnp.bfloat16)
    lo = (z - hi.astype(jnp.float32)).astype(jnp.bfloat16)
    return hi, lo


def _dot(a, b):
    return jax.lax.dot_general(a, b, (((1,), (0,)), ((), ())),
                               preferred_element_type=jnp.float32)


def _body(a_ref, low_ref, high_ref, fa_ref, r_ref, e_ref, kmod_ref, o_ref):
    a = a_ref[...]
    low = low_ref[...]
    high = high_ref[...]
    scale = 2.0 / (high - low)
    x = (a - low) * scale + 1.0
    x = jnp.clip(x, -1.0, 1.0)

    xh, xm, xl = _split3(x)
    xr = _dot(jnp.concatenate([xh, xm, xl], axis=1), r_ref[...])

    base = xr * fa_ref[...]
    s0 = jnp.sin(base)
    c0 = jnp.cos(base)

    s4, c4 = s0, c0
    for _ in range(4):
        s4, c4 = 2.0 * s4 * c4, 2.0 * c4 * c4 - 1.0

    E2 = e_ref[...]

    def expand(z):
        hi, lo = _split2(z)
        return _dot(jnp.concatenate([hi, lo], axis=1), E2)

    S0 = expand(s0)
    C0 = expand(c0)
    S4 = expand(s4)
    C4 = expand(c4)

    kmod = kmod_ref[...]
    k4 = jnp.where(kmod >= 4, kmod - 4, kmod)
    S = jnp.where(kmod >= 4, S4, S0)
    C = jnp.where(kmod >= 4, C4, C0)
    for j in range(1, 4):
        sd = 2.0 * S * C
        cd = 2.0 * C * C - 1.0
        act = k4 >= j
        S = jnp.where(act, sd, S)
        C = jnp.where(act, cd, C)

    o_ref[:, 0:2048] = S
    o_ref[:, 2048:4096] = C


def kernel(action, action_low, action_high, freqs):
    B, A = action.shape
    F = freqs.shape[0]
    P = A * _NCHAIN
    BB = 1024

    low2 = action_low.reshape(1, A)
    high2 = action_high.reshape(1, A)
    fa = jnp.tile(freqs[::_L], (A,)).reshape(1, P)

    a_idx = jnp.arange(A)[:, None]
    p_idx = jnp.arange(P)[None, :]
    R1 = (p_idx // _NCHAIN == a_idx).astype(jnp.bfloat16)
    R = jnp.concatenate([R1, R1, R1], axis=0)

    pp_idx = jnp.arange(P)[:, None]
    l_idx = jnp.arange(A * F)[None, :]
    E1 = (l_idx // _L == pp_idx).astype(jnp.bfloat16)
    E = jnp.concatenate([E1, E1], axis=0)
    kmod = (jnp.arange(A * F, dtype=jnp.int32) % _L).reshape(1, A * F)

    out = pl.pallas_call(
        _body,
        grid=(B // BB,),
        in_specs=[
            pl.BlockSpec((BB, A), lambda i: (i, 0)),
            pl.BlockSpec((1, A), lambda i: (0, 0)),
            pl.BlockSpec((1, A), lambda i: (0, 0)),
            pl.BlockSpec((1, P), lambda i: (0, 0)),
            pl.BlockSpec((3 * A, P), lambda i: (0, 0)),
            pl.BlockSpec((2 * P, A * F), lambda i: (0, 0)),
            pl.BlockSpec((1, A * F), lambda i: (0, 0)),
        ],
        out_specs=pl.BlockSpec((BB, 2 * A * F), lambda i: (i, 0)),
        out_shape=jax.ShapeDtypeStruct((B, 2 * A * F), jnp.float32),
        compiler_params=pltpu.CompilerParams(
            dimension_semantics=("parallel",),
        ),
    )(action, low2, high2, fa, R, E, kmod)

    return out


# speedup vs baseline: 1.2882x; 1.0253x over previous
import jax
import jax.numpy as jnp
from jax.experimental import pallas as pl
from jax.experimental.pallas import tpu as pltpu

_L = 8
_NCHAIN = 32 // _L


def _split3(z):
    hi = z.astype(jnp.bfloat16)
    r1 = z - hi.astype(jnp.float32)
    mid = r1.astype(jnp.bfloat16)
    lo = (r1 - mid.astype(jnp.float32)).astype(jnp.bfloat16)
    return hi, mid, lo


def _split2(z):
    hi = z.astype(jnp.bfloat16)
    lo = (z - hi.astype(jnp.float32)).astype(jnp.bfloat16)
    return hi, lo


def _dot(a, b):
    return jax.lax.dot_general(a, b, (((1,), (0,)), ((), ())),
                               preferred_element_type=jnp.float32)


def _body(a_ref, low_ref, high_ref, fa_ref, r_ref, e_ref, kmod_ref, o_ref):
    a = a_ref[...]
    low = low_ref[...]
    high = high_ref[...]
    scale = 2.0 / (high - low)
    x = (a - low) * scale + 1.0
    x = jnp.clip(x, -1.0, 1.0)

    xh, xm, xl = _split3(x)
    xr = _dot(jnp.concatenate([xh, xm, xl], axis=1), r_ref[...])

    base = xr * fa_ref[...]
    s0 = jnp.sin(base)
    c0 = jnp.cos(base)

    s4, c4 = s0, c0
    for _ in range(4):
        s4, c4 = 2.0 * s4 * c4, 2.0 * c4 * c4 - 1.0

    EF = e_ref[...]

    def expand(z0, z4):
        h0, l0 = _split2(z0)
        h4, l4 = _split2(z4)
        return _dot(jnp.concatenate([h0, l0, h4, l4], axis=1), EF)

    S = expand(s0, s4)
    C = expand(c0, c4)

    k4 = kmod_ref[...]
    for j in range(1, 4):
        sd = 2.0 * S * C
        cd = 2.0 * C * C - 1.0
        act = k4 >= j
        S = jnp.where(act, sd, S)
        C = jnp.where(act, cd, C)

    o_ref[:, 0:2048] = S
    o_ref[:, 2048:4096] = C


def kernel(action, action_low, action_high, freqs):
    B, A = action.shape
    F = freqs.shape[0]
    P = A * _NCHAIN
    BB = 1024

    low2 = action_low.reshape(1, A)
    high2 = action_high.reshape(1, A)
    fa = jnp.tile(freqs[::_L], (A,)).reshape(1, P)

    a_idx = jnp.arange(A)[:, None]
    p_idx = jnp.arange(P)[None, :]
    R1 = (p_idx // _NCHAIN == a_idx).astype(jnp.bfloat16)
    R = jnp.concatenate([R1, R1, R1], axis=0)

    pp_idx = jnp.arange(P)[:, None]
    l_idx = jnp.arange(A * F)[None, :]
    hit = l_idx // _L == pp_idx
    E0 = (hit & (l_idx % _L < 4)).astype(jnp.bfloat16)
    E4 = (hit & (l_idx % _L >= 4)).astype(jnp.bfloat16)
    E = jnp.concatenate([E0, E0, E4, E4], axis=0)
    kmod = ((jnp.arange(A * F, dtype=jnp.int32) % _L) % 4).reshape(1, A * F)

    out = pl.pallas_call(
        _body,
        grid=(B // BB,),
        in_specs=[
            pl.BlockSpec((BB, A), lambda i: (i, 0)),
            pl.BlockSpec((1, A), lambda i: (0, 0)),
            pl.BlockSpec((1, A), lambda i: (0, 0)),
            pl.BlockSpec((1, P), lambda i: (0, 0)),
            pl.BlockSpec((3 * A, P), lambda i: (0, 0)),
            pl.BlockSpec((4 * P, A * F), lambda i: (0, 0)),
            pl.BlockSpec((1, A * F), lambda i: (0, 0)),
        ],
        out_specs=pl.BlockSpec((BB, 2 * A * F), lambda i: (i, 0)),
        out_shape=jax.ShapeDtypeStruct((B, 2 * A * F), jnp.float32),
        compiler_params=pltpu.CompilerParams(
            dimension_semantics=("parallel",),
        ),
    )(action, low2, high2, fa, R, E, kmod)

    return out
